# baseline (speedup 1.0000x reference)
import jax
import jax.numpy as jnp
from jax import lax
from jax.experimental import pallas as pl
from jax.experimental.pallas import tpu as pltpu
from jax.experimental.pallas import tpu_sc as plsc

NS = 16
L = 16
B = 16384
BT = B // NS
V = 5000
D = 64
G = D // L
VP = 5120
CR = VP // D
RT = VP // NS
CRT = CR // NS
SCALE = 1.0 / B
ZERO16 = (0.0,) * L


def _acc_rows(cnt_vec, tbl_v, row0, nk, acc):
    acc = list(acc)
    for kk in range(nk):
        c = lax.broadcast(cnt_vec[kk], (L,))
        h = (kk % 2) * G
        for g in range(G):
            acc[h + g] = acc[h + g] + c * tbl_v[row0 + kk, pl.ds(g * L, L)]
    return tuple(acc)


def _sc_body(idx_hbm, table_hbm, out_hbm, idx_v, cnt_v, cntm_v, tbl_v,
             acc_v, part_v, shcnt_v, shacc_v, semt, semm):
    sid = lax.axis_index("s")
    nlast = V - RT * (NS - 1)
    @pl.when(sid < NS - 1)
    def _():
        pltpu.async_copy(table_hbm.at[pl.ds(sid * RT, RT)], tbl_v, semt)

    @pl.when(sid == NS - 1)
    def _():
        pltpu.async_copy(table_hbm.at[pl.ds((NS - 1) * RT, nlast)],
                         tbl_v.at[pl.ds(0, nlast)], semt)

    scope = jax.named_scope
    with scope("idx_stage"):
        pltpu.sync_copy(idx_hbm.at[sid], idx_v)

    def zbody(i, _):
        for j in range(G):
            cnt_v[i, pl.ds(j * L, L)] = jnp.zeros((L,), jnp.float32)
        return 0

    with scope("zero_cnt"):
        lax.fori_loop(0, CR, zbody, 0)
    ones = jnp.ones((L,), jnp.float32)

    def hbody(i, _):
        iv = idx_v[pl.ds(i * L, L)]
        r = lax.shift_right_logical(iv, 6)
        c = lax.bitwise_and(iv, D - 1)
        plsc.addupdate_scatter(cnt_v, [r, c], ones)
        return 0

    with scope("hist"):
        lax.fori_loop(0, BT // L, hbody, 0)
    with scope("publish_cnt"):
        pltpu.sync_copy(cnt_v, shcnt_v.at[sid])
    with scope("barrier1"):
        plsc.subcore_barrier()

    with scope("merge_dma"):
        mcopies = [
            pltpu.async_copy(shcnt_v.at[s, pl.ds(sid * CRT, CRT)],
                             cntm_v.at[s], semm)
            for s in range(NS)
        ]
        for c in mcopies:
            c.wait()
    with scope("merge_add"):
        for r in range(CRT):
            for j in range(G):
                s = cntm_v[0, r, pl.ds(j * L, L)]
                for t in range(1, NS):
                    s = s + cntm_v[t, r, pl.ds(j * L, L)]
                cnt_v[r, pl.ds(j * L, L)] = s

    acc0 = tuple(jnp.zeros((L,), jnp.float32) for _ in range(2 * G))

    @pl.when(sid < NS - 1)
    def _():
        with scope("tbl_wait"):
            pltpu.make_async_copy(
                table_hbm.at[pl.ds(sid * RT, RT)], tbl_v, semt).wait()

        def gbody(gi, a):
            cv = cnt_v[lax.div(gi, G), pl.ds(lax.rem(gi, G) * L, L)]
            return _acc_rows(cv, tbl_v, gi * L, L, a)

        with scope("wsum"):
            acc = lax.fori_loop(0, RT // L, gbody, acc0)
            for k in range(G):
                acc_v[pl.ds(k * L, L)] = acc[k] + acc[G + k]

    @pl.when(sid == NS - 1)
    def _():
        nfull = nlast // L
        ntail = nlast - nfull * L
        pltpu.make_async_copy(
            table_hbm.at[pl.ds((NS - 1) * RT, nlast)],
            tbl_v.at[pl.ds(0, nlast)], semt).wait()

        def gbody(gi, a):
            cv = cnt_v[lax.div(gi, G), pl.ds(lax.rem(gi, G) * L, L)]
            return _acc_rows(cv, tbl_v, gi * L, L, a)

        acc = lax.fori_loop(0, nfull, gbody, acc0)
        cv = cnt_v[nfull // G, pl.ds((nfull % G) * L, L)]
        acc = _acc_rows(cv, tbl_v, nfull * L, ntail, acc)
        for k in range(G):
            acc_v[pl.ds(k * L, L)] = acc[k] + acc[G + k]

    with scope("finalize"):
        pltpu.sync_copy(acc_v, shacc_v.at[sid])
        plsc.subcore_barrier()

        @pl.when(sid == 0)
        def _():
            pltpu.sync_copy(shacc_v, part_v)
            for k in range(G):
                s = part_v[0, pl.ds(k * L, L)]
                for t in range(1, NS):
                    s = s + part_v[t, pl.ds(k * L, L)]
                acc_v[pl.ds(k * L, L)] = s * SCALE
            pltpu.sync_copy(acc_v, out_hbm)


def kernel(idxs, table):
    idx2 = idxs.reshape(NS, BT)
    mesh = plsc.VectorSubcoreMesh(
        core_axis_name="c", subcore_axis_name="s", num_cores=1)
    f = pl.kernel(
        _sc_body,
        out_type=jax.ShapeDtypeStruct((D,), jnp.float32),
        mesh=mesh,
        scratch_types=[
            pltpu.VMEM((BT,), jnp.int32),
            pltpu.VMEM((CR, D), jnp.float32),
            pltpu.VMEM((NS, CRT, D), jnp.float32),
            pltpu.VMEM((RT, D), jnp.float32),
            pltpu.VMEM((D,), jnp.float32),
            pltpu.VMEM((NS, D), jnp.float32),
            pltpu.VMEM_SHARED((NS, CR, D), jnp.float32),
            pltpu.VMEM_SHARED((NS, D), jnp.float32),
            pltpu.SemaphoreType.DMA,
            pltpu.SemaphoreType.DMA,
        ],
        compiler_params=pltpu.CompilerParams(
            use_tc_tiling_on_sc=False, needs_layout_passes=False),
    )
    return f(idx2, table)


# speedup vs baseline: 1.0006x; 1.0006x over previous
import jax
import jax.numpy as jnp
from jax import lax
from jax.experimental import pallas as pl
from jax.experimental.pallas import tpu as pltpu
from jax.experimental.pallas import tpu_sc as plsc

NS = 16
L = 16
B = 16384
BT = B // NS
V = 5000
D = 64
G = D // L
VP = 5120
CR = VP // D
RT = VP // NS
CRT = CR // NS
NLAST = V - RT * (NS - 1)
SCALE = 1.0 / B


def _acc_rows(cnt_vec, tbl_v, row0, nk, acc):
    acc = list(acc)
    for kk in range(nk):
        c = lax.broadcast(cnt_vec[kk], (L,))
        h = (kk % 2) * G
        for g in range(G):
            acc[h + g] = acc[h + g] + c * tbl_v[row0 + kk, pl.ds(g * L, L)]
    return tuple(acc)


def _sc_body(idx_hbm, table_hbm, out_hbm, idx_v, cnt_v, cntm_v, tbl_v,
             acc_v, part_v, shcnt_v, shacc_v, semt, semm):
    sid = lax.axis_index("s")
    scope = jax.named_scope
    @pl.when(sid < NS - 1)
    def _():
        pltpu.async_copy(table_hbm.at[pl.ds(sid * RT, RT)], tbl_v, semt)

    @pl.when(sid == NS - 1)
    def _():
        pltpu.async_copy(table_hbm.at[pl.ds((NS - 1) * RT, NLAST)],
                         tbl_v.at[pl.ds(0, NLAST)], semt)

    with scope("idx_stage"):
        start = pl.multiple_of(sid * BT, BT)
        pltpu.sync_copy(idx_hbm.at[pl.ds(start, BT)], idx_v)

    def zbody(i, _):
        for j in range(G):
            cnt_v[i, pl.ds(j * L, L)] = jnp.zeros((L,), jnp.float32)
        return 0

    with scope("zero_cnt"):
        lax.fori_loop(0, CR, zbody, 0)
    ones = jnp.ones((L,), jnp.float32)

    def hbody(i, _):
        iv = idx_v[pl.ds(i * L, L)]
        r = lax.shift_right_logical(iv, 6)
        c = lax.bitwise_and(iv, D - 1)
        plsc.addupdate_scatter(cnt_v, [r, c], ones)
        return 0

    with scope("hist"):
        lax.fori_loop(0, BT // L, hbody, 0)
    with scope("publish_cnt"):
        pltpu.sync_copy(cnt_v, shcnt_v.at[sid])
    with scope("barrier1"):
        plsc.subcore_barrier()

    with scope("merge_dma"):
        mcopies = [
            pltpu.async_copy(shcnt_v.at[s, pl.ds(sid * CRT, CRT)],
                             cntm_v.at[s], semm)
            for s in range(NS)
        ]
        for c in mcopies:
            c.wait()
    with scope("merge_add"):
        for r in range(CRT):
            for j in range(G):
                s = cntm_v[0, r, pl.ds(j * L, L)]
                for t in range(1, NS):
                    s = s + cntm_v[t, r, pl.ds(j * L, L)]
                cnt_v[r, pl.ds(j * L, L)] = s

    acc0 = tuple(jnp.zeros((L,), jnp.float32) for _ in range(2 * G))

    @pl.when(sid < NS - 1)
    def _():
        with scope("tbl_wait"):
            pltpu.make_async_copy(
                table_hbm.at[pl.ds(sid * RT, RT)], tbl_v, semt).wait()

        def gbody(gi, a):
            cv = cnt_v[lax.div(gi, G), pl.ds(lax.rem(gi, G) * L, L)]
            return _acc_rows(cv, tbl_v, gi * L, L, a)

        with scope("wsum"):
            acc = lax.fori_loop(0, RT // L, gbody, acc0)
            for k in range(G):
                acc_v[pl.ds(k * L, L)] = acc[k] + acc[G + k]

    @pl.when(sid == NS - 1)
    def _():
        nfull = NLAST // L
        ntail = NLAST - nfull * L
        with scope("tbl_wait"):
            pltpu.make_async_copy(
                table_hbm.at[pl.ds((NS - 1) * RT, NLAST)],
                tbl_v.at[pl.ds(0, NLAST)], semt).wait()

        def gbody(gi, a):
            cv = cnt_v[lax.div(gi, G), pl.ds(lax.rem(gi, G) * L, L)]
            return _acc_rows(cv, tbl_v, gi * L, L, a)

        with scope("wsum"):
            acc = lax.fori_loop(0, nfull, gbody, acc0)
            cv = cnt_v[nfull // G, pl.ds((nfull % G) * L, L)]
            acc = _acc_rows(cv, tbl_v, nfull * L, ntail, acc)
            for k in range(G):
                acc_v[pl.ds(k * L, L)] = acc[k] + acc[G + k]

    with scope("finalize"):
        pltpu.sync_copy(acc_v, shacc_v.at[sid])
        plsc.subcore_barrier()

        @pl.when(sid == 0)
        def _():
            pltpu.sync_copy(shacc_v, part_v)
            for k in range(G):
                s = part_v[0, pl.ds(k * L, L)]
                for t in range(1, NS):
                    s = s + part_v[t, pl.ds(k * L, L)]
                acc_v[pl.ds(k * L, L)] = s * SCALE
            pltpu.sync_copy(acc_v, out_hbm)


def kernel(idxs, table):
    mesh = plsc.VectorSubcoreMesh(
        core_axis_name="c", subcore_axis_name="s", num_cores=1)
    f = pl.kernel(
        _sc_body,
        out_type=jax.ShapeDtypeStruct((D,), jnp.float32),
        mesh=mesh,
        scratch_types=[
            pltpu.VMEM((BT,), jnp.int32),
            pltpu.VMEM((CR, D), jnp.float32),
            pltpu.VMEM((NS, CRT, D), jnp.float32),
            pltpu.VMEM((RT, D), jnp.float32),
            pltpu.VMEM((D,), jnp.float32),
            pltpu.VMEM((NS, D), jnp.float32),
            pltpu.VMEM_SHARED((NS, CR, D), jnp.float32),
            pltpu.VMEM_SHARED((NS, D), jnp.float32),
            pltpu.SemaphoreType.DMA,
            pltpu.SemaphoreType.DMA,
        ],
        compiler_params=pltpu.CompilerParams(
            use_tc_tiling_on_sc=False, needs_layout_passes=False),
    )
    return f(idxs, table)


# speedup vs baseline: 1.0791x; 1.0784x over previous
import jax
import jax.numpy as jnp
from jax import lax
from jax.experimental import pallas as pl
from jax.experimental.pallas import tpu as pltpu
from jax.experimental.pallas import tpu_sc as plsc

NS = 16
L = 16
B = 16384
BT = B // NS
V = 5000
D = 64
WT = 8
FPT = D // WT
CRR = 48
NCH = V // L
NTL = V - NCH * L
SCALE = 1.0 / B


def _sc_body(idx_hbm, tableT_hbm, out_hbm, idx_v, cnt_v, cntm_v, tbl_v,
             acc_v, rows_v, shcnt_v, shacc_v, semt):
    sid = lax.axis_index("s")
    scope = jax.named_scope

    @pl.when(sid < WT)
    def _():
        fstart = pl.multiple_of(sid * FPT, FPT)
        pltpu.async_copy(tableT_hbm.at[pl.ds(fstart, FPT)], tbl_v, semt)

    with scope("idx_stage"):
        start = pl.multiple_of(sid * BT, BT)
        pltpu.sync_copy(idx_hbm.at[pl.ds(start, BT)], idx_v)

    def zbody(i, _):
        for j in range(8):
            cnt_v[i, pl.ds(j * L, L)] = jnp.zeros((L,), jnp.float32)
        return 0

    with scope("zero_cnt"):
        lax.fori_loop(0, CRR, zbody, 0)
        iota = lax.iota(jnp.int32, L)
        for k in range(CRR // L):
            rows_v[pl.ds(k * L, L)] = iota + k * L

    @pl.when(sid == 0)
    def _():
        with scope("init_shcnt"):
            pltpu.sync_copy(cnt_v, shcnt_v)

    ones = jnp.ones((L,), jnp.float32)

    def hbody(i, _):
        iv = idx_v[pl.ds(i * L, L)]
        r = lax.shift_right_logical(iv, 7)
        c = lax.bitwise_and(iv, 127)
        plsc.addupdate_scatter(cnt_v, [r, c], ones)
        return 0

    with scope("hist"):
        lax.fori_loop(0, BT // L, hbody, 0)
    with scope("barrier0"):
        plsc.subcore_barrier()
    with scope("merge_add"):
        pltpu.sync_copy(cnt_v, shcnt_v.at[rows_v], add=True)
    with scope("barrier1"):
        plsc.subcore_barrier()

    @pl.when(sid < WT)
    def _():
        with scope("cnt_fetch"):
            pltpu.sync_copy(shcnt_v, cntm_v)
        with scope("tbl_wait"):
            fstart = pl.multiple_of(sid * FPT, FPT)
            pltpu.make_async_copy(
                tableT_hbm.at[pl.ds(fstart, FPT)], tbl_v, semt).wait()

        def gbody(gi, a):
            cv = cntm_v[lax.shift_right_logical(gi, 3),
                        pl.ds(lax.bitwise_and(gi, 7) * L, L)]
            col = gi * L
            return tuple(
                a[f] + cv * tbl_v[f, pl.ds(col, L)] for f in range(FPT))

        with scope("wsum"):
            acc0 = tuple(jnp.zeros((L,), jnp.float32) for _ in range(FPT))
            acc = lax.fori_loop(0, NCH, gbody, acc0)
            lane = lax.iota(jnp.int32, L)
            cv = cntm_v[NCH // 8, pl.ds((NCH % 8) * L, L)]
            cvt = jnp.where(lane < NTL, cv, 0.0)
            tcol = NCH * L + (lane & (NTL - 1))
            acc = tuple(
                a + cvt * plsc.load_gather(
                    tbl_v, [jnp.full((L,), f, jnp.int32), tcol])
                for f, a in enumerate(acc))
            lane = lax.iota(jnp.int32, L)
            res = jnp.zeros((L,), jnp.float32)
            for f in range(FPT):
                tot = lax.broadcast(jnp.sum(acc[f], axis=0), (L,))
                res = jnp.where(lane == f, tot, res)
            acc_v[pl.ds(0, L)] = res
            pltpu.sync_copy(acc_v.at[pl.ds(0, FPT)],
                            shacc_v.at[pl.ds(sid * FPT, FPT)])

    with scope("finalize"):
        plsc.subcore_barrier()

        @pl.when(sid == 0)
        def _():
            pltpu.sync_copy(shacc_v, acc_v)
            for k in range(D // L):
                acc_v[pl.ds(k * L, L)] = acc_v[pl.ds(k * L, L)] * SCALE
            pltpu.sync_copy(acc_v, out_hbm)


def kernel(idxs, table):
    mesh = plsc.VectorSubcoreMesh(
        core_axis_name="c", subcore_axis_name="s", num_cores=1)
    f = pl.kernel(
        _sc_body,
        out_type=jax.ShapeDtypeStruct((D,), jnp.float32),
        mesh=mesh,
        scratch_types=[
            pltpu.VMEM((BT,), jnp.int32),
            pltpu.VMEM((CRR, 128), jnp.float32),
            pltpu.VMEM((CRR, 128), jnp.float32),
            pltpu.VMEM((FPT, V), jnp.float32),
            pltpu.VMEM((D,), jnp.float32),
            pltpu.VMEM((CRR,), jnp.int32),
            pltpu.VMEM_SHARED((CRR, 128), jnp.float32),
            pltpu.VMEM_SHARED((D,), jnp.float32),
            pltpu.SemaphoreType.DMA,
        ],
        compiler_params=pltpu.CompilerParams(needs_layout_passes=False),
    )
    return f(idxs, table.T)


# speedup vs baseline: 1.1027x; 1.0219x over previous
import jax
import jax.numpy as jnp
from jax import lax
from jax.experimental import pallas as pl
from jax.experimental.pallas import tpu as pltpu
from jax.experimental.pallas import tpu_sc as plsc

NS = 16
L = 16
B = 16384
BT = B // NS
V = 5000
D = 64
WT = 8
FPT = D // WT
CRR = 48
NCH = V // L
NTL = V - NCH * L
SCALE = 1.0 / B


def _sc_body(idx_hbm, tableT_hbm, out_hbm, idx_v, cnt_v, cntm_v, tbl_v,
             acc_v, rows_v, fin_v, shcnt_v, shacc_v, semt, semi):
    sid = lax.axis_index("s")
    fgrp = lax.bitwise_and(sid, WT - 1)
    half = lax.shift_right_logical(sid, 3)
    scope = jax.named_scope

    start = pl.multiple_of(sid * BT, BT)
    pltpu.async_copy(idx_hbm.at[pl.ds(start, BT)], idx_v, semi)
    fstart = pl.multiple_of(fgrp * FPT, FPT)
    pltpu.async_copy(tableT_hbm.at[pl.ds(fstart, FPT)], tbl_v, semt)

    def zbody(i, _):
        for j in range(8):
            cnt_v[i, pl.ds(j * L, L)] = jnp.zeros((L,), jnp.float32)
        return 0

    with scope("zero_cnt"):
        lax.fori_loop(0, CRR, zbody, 0)
        iota = lax.iota(jnp.int32, L)
        for k in range(CRR // L):
            rows_v[pl.ds(k * L, L)] = iota + k * L

    @pl.when(sid == 0)
    def _():
        with scope("init_shcnt"):
            pltpu.sync_copy(cnt_v, shcnt_v)

    ones = jnp.ones((L,), jnp.float32)
    with scope("idx_wait"):
        pltpu.make_async_copy(
            idx_hbm.at[pl.ds(start, BT)], idx_v, semi).wait()

    def hbody(i, _):
        iv = idx_v[pl.ds(i * L, L)]
        r = lax.shift_right_logical(iv, 7)
        c = lax.bitwise_and(iv, 127)
        plsc.addupdate_scatter(cnt_v, [r, c], ones)
        return 0

    with scope("hist"):
        lax.fori_loop(0, BT // L, hbody, 0)
    with scope("barrier0"):
        plsc.subcore_barrier()
    with scope("merge_add"):
        pltpu.sync_copy(cnt_v, shcnt_v.at[rows_v], add=True)
    with scope("barrier1"):
        plsc.subcore_barrier()

    with scope("cnt_fetch"):
        pltpu.sync_copy(shcnt_v, cntm_v)
    with scope("tbl_wait"):
        pltpu.make_async_copy(
            tableT_hbm.at[pl.ds(fstart, FPT)], tbl_v, semt).wait()

    def gbody(gi, a):
        cv = cntm_v[lax.shift_right_logical(gi, 3),
                    pl.ds(lax.bitwise_and(gi, 7) * L, L)]
        col = gi * L
        return tuple(
            a[f] + cv * tbl_v[f, pl.ds(col, L)] for f in range(FPT))

    with scope("wsum"):
        acc0 = tuple(jnp.zeros((L,), jnp.float32) for _ in range(FPT))
        g0 = half * (NCH // 2)
        acc = lax.fori_loop(g0, g0 + NCH // 2, gbody, acc0)
        lane = lax.iota(jnp.int32, L)
        cv = cntm_v[NCH // 8, pl.ds((NCH % 8) * L, L)]
        cvt = jnp.where((lane < NTL) & (half == 1), cv, 0.0)
        tcol = NCH * L + (lane & (NTL - 1))
        acc = tuple(
            a + cvt * plsc.load_gather(
                tbl_v, [jnp.full((L,), f, jnp.int32), tcol])
            for f, a in enumerate(acc))
        res = jnp.zeros((L,), jnp.float32)
        for f in range(FPT):
            tot = lax.broadcast(jnp.sum(acc[f], axis=0), (L,))
            res = jnp.where(lane == f, tot, res)
        acc_v[pl.ds(0, L)] = res
        pltpu.sync_copy(acc_v.at[pl.ds(0, FPT)],
                        shacc_v.at[half, pl.ds(fgrp * FPT, FPT)])

    with scope("finalize"):
        plsc.subcore_barrier()

        @pl.when(sid == 0)
        def _():
            pltpu.sync_copy(shacc_v, fin_v)
            for k in range(D // L):
                s = fin_v[0, pl.ds(k * L, L)] + fin_v[1, pl.ds(k * L, L)]
                acc_v[pl.ds(k * L, L)] = s * SCALE
            pltpu.sync_copy(acc_v, out_hbm)


def kernel(idxs, table):
    mesh = plsc.VectorSubcoreMesh(
        core_axis_name="c", subcore_axis_name="s", num_cores=1)
    f = pl.kernel(
        _sc_body,
        out_type=jax.ShapeDtypeStruct((D,), jnp.float32),
        mesh=mesh,
        scratch_types=[
            pltpu.VMEM((BT,), jnp.int32),
            pltpu.VMEM((CRR, 128), jnp.float32),
            pltpu.VMEM((CRR, 128), jnp.float32),
            pltpu.VMEM((FPT, V), jnp.float32),
            pltpu.VMEM((D,), jnp.float32),
            pltpu.VMEM((CRR,), jnp.int32),
            pltpu.VMEM((2, D), jnp.float32),
            pltpu.VMEM_SHARED((CRR, 128), jnp.float32),
            pltpu.VMEM_SHARED((2, D), jnp.float32),
            pltpu.SemaphoreType.DMA,
            pltpu.SemaphoreType.DMA,
        ],
        compiler_params=pltpu.CompilerParams(needs_layout_passes=False),
    )
    return f(idxs, table.T)
